# baseline (speedup 1.0000x reference)
import functools

import jax
import jax.numpy as jnp
from jax.experimental import pallas as pl
from jax.experimental.pallas import tpu as pltpu

_N = 20000
_B = 128
_NROW = 160
_K_FULL = 4096
_K_FAST = 256
_LROWS = 128
_LCOLS = 157
_M = 8
_POOL = _LROWS * _M
_POOL_ROWS = _POOL // _B
_POST = 100
_OUT_ROWS = 104
_NMS_T = 0.7
_SCORE_T = 0.05
_IMG = 1024.0


def _colify(v):
  return jnp.broadcast_to(v, (_B, _B)).T


def _pair_suppress(ax1, ay1, ax2, ay2, aarea, bx1, by1, bx2, by2, barea):
  cx1 = _colify(ax1)
  cy1 = _colify(ay1)
  cx2 = _colify(ax2)
  cy2 = _colify(ay2)
  carea = _colify(aarea)
  lt_x = jnp.maximum(cx1, bx1)
  lt_y = jnp.maximum(cy1, by1)
  rb_x = jnp.minimum(cx2, bx2)
  rb_y = jnp.minimum(cy2, by2)
  w = jnp.maximum(rb_x - lt_x, 0.0)
  h = jnp.maximum(rb_y - lt_y, 0.0)
  inter = w * h
  union = jnp.maximum(carea + barea - inter, 1e-8)
  return (inter > _NMS_T * union).astype(jnp.float32)


def _make_nms_body(nblk, fast):
  def _nms_body(*refs):
    if fast:
      (fx1_ref, fy1_ref, fx2_ref, fy2_ref, s_ref, idx_ref, ftab_ref,
       tp_ref, out_ref, cnt_ref,
       keep_ref, cx1_ref, cy1_ref, cx2_ref, cy2_ref) = refs
    else:
      (fx1_ref, fy1_ref, fx2_ref, fy2_ref, s_ref, idx_ref,
       out_ref, cnt_ref,
       keep_ref, cx1_ref, cy1_ref, cx2_ref, cy2_ref) = refs
    keep_ref[...] = jnp.zeros((nblk, _B), jnp.float32)
    cx1_ref[...] = jnp.zeros((nblk, _B), jnp.float32)
    cy1_ref[...] = jnp.zeros((nblk, _B), jnp.float32)
    cx2_ref[...] = jnp.zeros((nblk, _B), jnp.float32)
    cy2_ref[...] = jnp.zeros((nblk, _B), jnp.float32)

    def row(ref, i):
      return ref[pl.ds(i, 1), :]

    def onehot_gather(ref, nrows, rsel, lsel):
      lane_iota = jax.lax.broadcasted_iota(jnp.int32, (_B, _B), 0)
      loh = (lane_iota == lsel).astype(jnp.float32)
      row_iota = jax.lax.broadcasted_iota(jnp.int32, (nrows, _B), 0)
      roh = (row_iota == rsel).astype(jnp.float32)
      c2 = jax.lax.dot(ref[...], loh,
                       precision=jax.lax.Precision.HIGHEST)
      return jnp.sum(c2 * roh, axis=0, keepdims=True)

    def block_body(carry):
      i, cnt = carry
      if fast:
        posr = row(idx_ref, i)
        idxf = onehot_gather(ftab_ref, _POOL_ROWS, posr >> 7, posr & 127)
        idxr = idxf.astype(jnp.int32)
      else:
        idxr = row(idx_ref, i)
      rsel = idxr >> 7
      lsel = idxr & 127

      bx1 = onehot_gather(fx1_ref, _NROW, rsel, lsel)
      by1 = onehot_gather(fy1_ref, _NROW, rsel, lsel)
      bx2 = onehot_gather(fx2_ref, _NROW, rsel, lsel)
      by2 = onehot_gather(fy2_ref, _NROW, rsel, lsel)
      bs = row(s_ref, i)
      barea = (bx2 - bx1) * (by2 - by1)
      cx1_ref[pl.ds(i, 1), :] = bx1
      cy1_ref[pl.ds(i, 1), :] = by1
      cx2_ref[pl.ds(i, 1), :] = bx2
      cy2_ref[pl.ds(i, 1), :] = by2

      def jbody(j, sup):
        jx1 = row(cx1_ref, j)
        jy1 = row(cy1_ref, j)
        jx2 = row(cx2_ref, j)
        jy2 = row(cy2_ref, j)
        jarea = (jx2 - jx1) * (jy2 - jy1)
        m = _pair_suppress(jx1, jy1, jx2, jy2, jarea,
                           bx1, by1, bx2, by2, barea)
        kj = _colify(row(keep_ref, j))
        return jnp.maximum(sup, jnp.max(m * kj, axis=0, keepdims=True))

      sup0 = jnp.zeros((1, _B), jnp.float32)
      if fast:
        sup = sup0
        for j in range(i):
          sup = jbody(j, sup)
      else:
        sup = jax.lax.fori_loop(0, i, jbody, sup0)

      m_ii = _pair_suppress(bx1, by1, bx2, by2, barea,
                            bx1, by1, bx2, by2, barea)
      r_idx = jax.lax.broadcasted_iota(jnp.int32, (_B, _B), 0)
      c_idx = jax.lax.broadcasted_iota(jnp.int32, (_B, _B), 1)
      m_ii = m_ii * (r_idx < c_idx).astype(jnp.float32)

      v = (bs > 0.0).astype(jnp.float32) * (1.0 - sup)

      def fix_cond(c):
        _, changed = c
        return changed

      def fix_body(c):
        k, _ = c
        kc = _colify(k)
        intra = jnp.max(m_ii * kc, axis=0, keepdims=True)
        knew = v * (1.0 - intra)
        return knew, jnp.any(knew != k)

      k, _ = jax.lax.while_loop(fix_cond, fix_body,
                                (v, jnp.bool_(True)))
      keep_ref[pl.ds(i, 1), :] = k
      return i + 1, cnt + jnp.sum(k)

    def block_cond(carry):
      i, cnt = carry
      return jnp.logical_and(i < nblk, cnt < float(_POST))

    if fast:
      carry = (0, jnp.float32(0.0))
      for i in range(nblk):
        carry = block_body(carry)
      _, cnt = carry
    else:
      _, cnt = jax.lax.while_loop(block_cond, block_body,
                                  (jnp.int32(0), jnp.float32(0.0)))
    if fast:
      qmask = (s_ref[...] > tp_ref[0, 0]).astype(jnp.float32)
      cnt = jnp.sum(keep_ref[...] * qmask)
    cnt_ref[...] = jnp.full((1, 1), cnt, jnp.float32)

    keep = keep_ref[...]
    nkeep = 1.0 - keep
    lt = (jax.lax.broadcasted_iota(jnp.int32, (_B, _B), 0)
          <= jax.lax.broadcasted_iota(jnp.int32, (_B, _B), 1)
          ).astype(jnp.float32)
    slt = (jax.lax.broadcasted_iota(jnp.int32, (nblk, nblk), 0)
           > jax.lax.broadcasted_iota(jnp.int32, (nblk, nblk), 1)
           ).astype(jnp.float32)

    def full_cumsum(x):
      rowc = jax.lax.dot(x, lt)
      rowtot = jnp.sum(x, axis=1, keepdims=True)
      carry = jax.lax.dot(slt, rowtot)
      return rowc + carry

    kcum = full_cumsum(keep)
    ncum = full_cumsum(nkeep)
    total_k = jnp.sum(keep)
    rank = keep * (kcum - 1.0) + nkeep * (total_k + ncum - 1.0)

    oidx = jax.lax.broadcasted_iota(jnp.int32, (_OUT_ROWS, 1), 0).astype(
        jnp.float32)
    cols = []
    for ref in (cx1_ref, cy1_ref, cx2_ref, cy2_ref, None):
      acc = jnp.zeros((_OUT_ROWS, 1), jnp.float32)
      for r in range(nblk):
        rr = rank[r:r + 1, :]
        oh = (oidx == rr).astype(jnp.float32)
        if ref is None:
          val = s_ref[r:r + 1, :] * keep[r:r + 1, :]
        else:
          val = ref[r:r + 1, :]
        acc = acc + jnp.sum(oh * val, axis=1, keepdims=True)
      cols.append(acc)
    cols.append(jnp.zeros((_OUT_ROWS, 3), jnp.float32))
    out_ref[...] = jnp.concatenate(cols, axis=1)

  return _nms_body


def _run_slow(table, scores_t, interpret):
  nblk = _K_FULL // _B
  top_scores, idx = jax.lax.top_k(scores_t, _K_FULL)
  s = top_scores.reshape(nblk, _B)
  idxb = idx.astype(jnp.int32).reshape(nblk, _B)
  out, _ = pl.pallas_call(
      _make_nms_body(nblk, fast=False),
      out_shape=(jax.ShapeDtypeStruct((_OUT_ROWS, 8), jnp.float32),
                 jax.ShapeDtypeStruct((1, 1), jnp.float32)),
      scratch_shapes=[pltpu.VMEM((nblk, _B), jnp.float32)] * 5,
      interpret=interpret,
  )(table[0], table[1], table[2], table[3], s, idxb)
  return out


def _run_fast(table, scores_t, interpret):
  nblk = _K_FAST // _B
  sp = jnp.pad(scores_t, (0, _LROWS * _LCOLS - _N))
  s2d = sp.reshape(_LROWS, _LCOLS)
  v8, r8 = jax.lax.top_k(s2d, _M)
  tprime = jnp.max(v8[:, _M - 1]).reshape(1, 1)
  flat = (jax.lax.broadcasted_iota(jnp.int32, (_LROWS, _M), 0) * _LCOLS
          + r8).astype(jnp.float32)
  ftab = flat.reshape(_POOL_ROWS, _B)
  s256, pos = jax.lax.top_k(v8.reshape(_POOL), _K_FAST)
  s = s256.reshape(nblk, _B)
  posb = pos.astype(jnp.int32).reshape(nblk, _B)
  out, cnt = pl.pallas_call(
      _make_nms_body(nblk, fast=True),
      out_shape=(jax.ShapeDtypeStruct((_OUT_ROWS, 8), jnp.float32),
                 jax.ShapeDtypeStruct((1, 1), jnp.float32)),
      scratch_shapes=[pltpu.VMEM((nblk, _B), jnp.float32)] * 5,
      interpret=interpret,
  )(table[0], table[1], table[2], table[3], s, posb, ftab, tprime)
  return out, cnt


@functools.partial(jax.jit, static_argnames=("interpret",))
def kernel(boxes, scores, interpret=False):
  boxes = jnp.clip(boxes, 0.0, _IMG)
  scores_t = jnp.where(scores >= _SCORE_T, scores, 0.0)

  table = jnp.pad(boxes, ((0, _NROW * _B - _N), (0, 0))).T
  table = table.reshape(4, _NROW, _B)

  out_fast, cnt = _run_fast(table, scores_t, interpret)

  def fast(_):
    return out_fast

  def slow(_):
    return _run_slow(table, scores_t, interpret)

  out = jax.lax.cond(cnt[0, 0] >= float(_POST), fast, slow, None)
  return out[:_POST, :5]


# speedup vs baseline: 1.0587x; 1.0587x over previous
import functools

import jax
import jax.numpy as jnp
from jax.experimental import pallas as pl
from jax.experimental.pallas import tpu as pltpu

_N = 20000
_B = 128
_NROW = 160
_K_FULL = 4096
_K_FAST = 256
_LROWS = 128
_LCOLS = 157
_M = 8
_POOL = _LROWS * _M
_POOL_ROWS = _POOL // _B
_POST = 100
_OUT_ROWS = 104
_NMS_T = 0.7
_SCORE_T = 0.05
_IMG = 1024.0


def _colify(v):
  return jnp.broadcast_to(v, (_B, _B)).T


def _pair_suppress(ax1, ay1, ax2, ay2, aarea, bx1, by1, bx2, by2, barea):
  cx1 = _colify(ax1)
  cy1 = _colify(ay1)
  cx2 = _colify(ax2)
  cy2 = _colify(ay2)
  carea = _colify(aarea)
  lt_x = jnp.maximum(cx1, bx1)
  lt_y = jnp.maximum(cy1, by1)
  rb_x = jnp.minimum(cx2, bx2)
  rb_y = jnp.minimum(cy2, by2)
  w = jnp.maximum(rb_x - lt_x, 0.0)
  h = jnp.maximum(rb_y - lt_y, 0.0)
  inter = w * h
  union = jnp.maximum(carea + barea - inter, 1e-8)
  return (inter > _NMS_T * union).astype(jnp.float32)


def _make_nms_body(nblk, fast):
  def _nms_body(*refs):
    if fast:
      (fx1_ref, fy1_ref, fx2_ref, fy2_ref, s_ref, idx_ref, ftab_ref,
       tp_ref, out_ref, cnt_ref,
       keep_ref, cx1_ref, cy1_ref, cx2_ref, cy2_ref) = refs
    else:
      (fx1_ref, fy1_ref, fx2_ref, fy2_ref, s_ref, idx_ref,
       out_ref, cnt_ref,
       keep_ref, cx1_ref, cy1_ref, cx2_ref, cy2_ref) = refs
    keep_ref[...] = jnp.zeros((nblk, _B), jnp.float32)
    cx1_ref[...] = jnp.zeros((nblk, _B), jnp.float32)
    cy1_ref[...] = jnp.zeros((nblk, _B), jnp.float32)
    cx2_ref[...] = jnp.zeros((nblk, _B), jnp.float32)
    cy2_ref[...] = jnp.zeros((nblk, _B), jnp.float32)

    def row(ref, i):
      return ref[pl.ds(i, 1), :]

    def onehot_gather(ref, nrows, rsel, lsel):
      lane_iota = jax.lax.broadcasted_iota(jnp.int32, (_B, _B), 0)
      loh = (lane_iota == lsel).astype(jnp.float32)
      row_iota = jax.lax.broadcasted_iota(jnp.int32, (nrows, _B), 0)
      roh = (row_iota == rsel).astype(jnp.float32)
      c2 = jax.lax.dot(ref[...], loh,
                       precision=jax.lax.Precision.HIGHEST)
      return jnp.sum(c2 * roh, axis=0, keepdims=True)

    def block_body(carry):
      i, cnt = carry
      if fast:
        posr = row(idx_ref, i)
        idxf = onehot_gather(ftab_ref, _POOL_ROWS, posr >> 7, posr & 127)
        idxr = idxf.astype(jnp.int32)
      else:
        idxr = row(idx_ref, i)
      rsel = idxr >> 7
      lsel = idxr & 127

      bx1 = onehot_gather(fx1_ref, _NROW, rsel, lsel)
      by1 = onehot_gather(fy1_ref, _NROW, rsel, lsel)
      bx2 = onehot_gather(fx2_ref, _NROW, rsel, lsel)
      by2 = onehot_gather(fy2_ref, _NROW, rsel, lsel)
      bs = row(s_ref, i)
      barea = (bx2 - bx1) * (by2 - by1)
      cx1_ref[pl.ds(i, 1), :] = bx1
      cy1_ref[pl.ds(i, 1), :] = by1
      cx2_ref[pl.ds(i, 1), :] = bx2
      cy2_ref[pl.ds(i, 1), :] = by2

      def jbody(j, sup):
        jx1 = row(cx1_ref, j)
        jy1 = row(cy1_ref, j)
        jx2 = row(cx2_ref, j)
        jy2 = row(cy2_ref, j)
        jarea = (jx2 - jx1) * (jy2 - jy1)
        m = _pair_suppress(jx1, jy1, jx2, jy2, jarea,
                           bx1, by1, bx2, by2, barea)
        kj = _colify(row(keep_ref, j))
        return jnp.maximum(sup, jnp.max(m * kj, axis=0, keepdims=True))

      sup = jax.lax.fori_loop(0, i, jbody, jnp.zeros((1, _B), jnp.float32))

      m_ii = _pair_suppress(bx1, by1, bx2, by2, barea,
                            bx1, by1, bx2, by2, barea)
      r_idx = jax.lax.broadcasted_iota(jnp.int32, (_B, _B), 0)
      c_idx = jax.lax.broadcasted_iota(jnp.int32, (_B, _B), 1)
      m_ii = m_ii * (r_idx < c_idx).astype(jnp.float32)

      v = (bs > 0.0).astype(jnp.float32) * (1.0 - sup)

      def fix_cond(c):
        _, changed = c
        return changed

      def fix_body(c):
        k, _ = c
        kc = _colify(k)
        intra = jnp.max(m_ii * kc, axis=0, keepdims=True)
        knew = v * (1.0 - intra)
        return knew, jnp.any(knew != k)

      k, _ = jax.lax.while_loop(fix_cond, fix_body,
                                (v, jnp.bool_(True)))
      keep_ref[pl.ds(i, 1), :] = k
      return i + 1, cnt + jnp.sum(k)

    def block_cond(carry):
      i, cnt = carry
      return jnp.logical_and(i < nblk, cnt < float(_POST))

    _, cnt = jax.lax.while_loop(block_cond, block_body,
                                (jnp.int32(0), jnp.float32(0.0)))
    if fast:
      qmask = (s_ref[...] > tp_ref[0, 0]).astype(jnp.float32)
      cnt = jnp.sum(keep_ref[...] * qmask)
    cnt_ref[...] = jnp.full((1, 1), cnt, jnp.float32)

    keep = keep_ref[...]
    nkeep = 1.0 - keep
    lt = (jax.lax.broadcasted_iota(jnp.int32, (_B, _B), 0)
          <= jax.lax.broadcasted_iota(jnp.int32, (_B, _B), 1)
          ).astype(jnp.float32)
    slt = (jax.lax.broadcasted_iota(jnp.int32, (nblk, nblk), 0)
           > jax.lax.broadcasted_iota(jnp.int32, (nblk, nblk), 1)
           ).astype(jnp.float32)

    def full_cumsum(x):
      rowc = jax.lax.dot(x, lt)
      rowtot = jnp.sum(x, axis=1, keepdims=True)
      carry = jax.lax.dot(slt, rowtot)
      return rowc + carry

    kcum = full_cumsum(keep)
    ncum = full_cumsum(nkeep)
    total_k = jnp.sum(keep)
    rank = keep * (kcum - 1.0) + nkeep * (total_k + ncum - 1.0)

    oidx = jax.lax.broadcasted_iota(jnp.int32, (_OUT_ROWS, 1), 0).astype(
        jnp.float32)
    cols = []
    for ref in (cx1_ref, cy1_ref, cx2_ref, cy2_ref, None):
      acc = jnp.zeros((_OUT_ROWS, 1), jnp.float32)
      for r in range(nblk):
        rr = rank[r:r + 1, :]
        oh = (oidx == rr).astype(jnp.float32)
        if ref is None:
          val = s_ref[r:r + 1, :] * keep[r:r + 1, :]
        else:
          val = ref[r:r + 1, :]
        acc = acc + jnp.sum(oh * val, axis=1, keepdims=True)
      cols.append(acc)
    cols.append(jnp.zeros((_OUT_ROWS, 3), jnp.float32))
    out_ref[...] = jnp.concatenate(cols, axis=1)

  return _nms_body


def _run_slow(table, scores_t, interpret):
  nblk = _K_FULL // _B
  top_scores, idx = jax.lax.top_k(scores_t, _K_FULL)
  s = top_scores.reshape(nblk, _B)
  idxb = idx.astype(jnp.int32).reshape(nblk, _B)
  out, _ = pl.pallas_call(
      _make_nms_body(nblk, fast=False),
      out_shape=(jax.ShapeDtypeStruct((_OUT_ROWS, 8), jnp.float32),
                 jax.ShapeDtypeStruct((1, 1), jnp.float32)),
      scratch_shapes=[pltpu.VMEM((nblk, _B), jnp.float32)] * 5,
      interpret=interpret,
  )(table[0], table[1], table[2], table[3], s, idxb)
  return out


def _run_fast(table, scores_t, interpret):
  nblk = _K_FAST // _B
  sp = jnp.pad(scores_t, (0, _LROWS * _LCOLS - _N))
  s2d = sp.reshape(_LROWS, _LCOLS)
  v8, r8 = jax.lax.top_k(s2d, _M)
  tprime = jnp.max(v8[:, _M - 1]).reshape(1, 1)
  flat = (jax.lax.broadcasted_iota(jnp.int32, (_LROWS, _M), 0) * _LCOLS
          + r8).astype(jnp.float32)
  ftab = flat.reshape(_POOL_ROWS, _B)
  s256, pos = jax.lax.top_k(v8.reshape(_POOL), _K_FAST)
  s = s256.reshape(nblk, _B)
  posb = pos.astype(jnp.int32).reshape(nblk, _B)
  out, cnt = pl.pallas_call(
      _make_nms_body(nblk, fast=True),
      out_shape=(jax.ShapeDtypeStruct((_OUT_ROWS, 8), jnp.float32),
                 jax.ShapeDtypeStruct((1, 1), jnp.float32)),
      scratch_shapes=[pltpu.VMEM((nblk, _B), jnp.float32)] * 5,
      interpret=interpret,
  )(table[0], table[1], table[2], table[3], s, posb, ftab, tprime)
  return out, cnt


@functools.partial(jax.jit, static_argnames=("interpret",))
def kernel(boxes, scores, interpret=False):
  boxes = jnp.clip(boxes, 0.0, _IMG)
  scores_t = jnp.where(scores >= _SCORE_T, scores, 0.0)

  table = jnp.pad(boxes, ((0, _NROW * _B - _N), (0, 0))).T
  table = table.reshape(4, _NROW, _B)

  out_fast, cnt = _run_fast(table, scores_t, interpret)

  def fast(_):
    return out_fast

  def slow(_):
    return _run_slow(table, scores_t, interpret)

  out = jax.lax.cond(cnt[0, 0] >= float(_POST), fast, slow, None)
  return out[:_POST, :5]
